# baseline (speedup 1.0000x reference)
import functools

import jax
import jax.numpy as jnp
from jax import lax
from jax.experimental import pallas as pl
from jax.experimental.pallas import tpu as pltpu
from jax.experimental.pallas import tpu_sc as plsc

CH = 128
F = 32
B = 1024
K = 50
N = B * K
N0 = N // 2
N1 = N - N0
NUM_RES = 4
PAD = 1024
BLK = 1024
NB0 = N0 // BLK
NBD = N // BLK
NBT = (N + PAD) // BLK

NTILES = 32
UB = 8
SLOTS = N // 16
NV = N // 16
PCHUNK = N // NTILES
GC = 80
NGC = PCHUNK // GC


def _enc_body(x0_ref, x1_ref, win_ref, bin_ref, wres_ref, bres_ref, wout_ref,
              bout_ref, out_ref):
    pid = pl.program_id(0)
    x = jnp.where(pid < NB0, x0_ref[...], x1_ref[...])
    x = jnp.where(jnp.isnan(x), 0.0, x)
    x = jnp.where(x == jnp.inf, 1e6, x)
    x = jnp.where(x == -jnp.inf, -1e6, x)
    dot = functools.partial(jnp.dot, preferred_element_type=jnp.float32)
    w_in = win_ref[0]
    h = jax.nn.relu(
        lax.dot_general(x, w_in, (((0,), (0,)), ((), ())),
                        preferred_element_type=jnp.float32) + bin_ref[0, 0])
    for i in range(NUM_RES):
        h = h + jax.nn.relu(dot(h, wres_ref[0, i]) + bres_ref[0, i])
    y = dot(h, wout_ref[0]) + bout_ref[0, 0]
    out_ref[...] = jnp.where(pid >= NBD, jnp.zeros_like(y), y)


def _encode_all(x0, x1, W_in, b_in, W_res, b_res, W_out, b_out):
    t = lambda i: (i >= NB0).astype(jnp.int32)
    return pl.pallas_call(
        _enc_body,
        grid=(NBT,),
        in_specs=[
            pl.BlockSpec((F, BLK), lambda i: (0, jnp.minimum(i, NB0 - 1))),
            pl.BlockSpec((F, BLK),
                         lambda i: (0, jnp.clip(i - NB0, 0, NBD - NB0 - 1))),
            pl.BlockSpec((1, F, CH), lambda i: (t(i), 0, 0)),
            pl.BlockSpec((1, 1, CH), lambda i: (t(i), 0, 0)),
            pl.BlockSpec((1, NUM_RES, CH, CH), lambda i: (t(i), 0, 0, 0)),
            pl.BlockSpec((1, NUM_RES, CH), lambda i: (t(i), 0, 0)),
            pl.BlockSpec((1, CH, CH), lambda i: (t(i), 0, 0)),
            pl.BlockSpec((1, 1, CH), lambda i: (t(i), 0, 0)),
        ],
        out_specs=pl.BlockSpec((BLK, CH), lambda i: (i, 0)),
        out_shape=jax.ShapeDtypeStruct((N + PAD, CH), jnp.float32),
    )(x0, x1, W_in, b_in.reshape(2, 1, CH), W_res, b_res, W_out,
      b_out.reshape(2, 1, CH))


def _winners_body(fb_hbm, fn_hbm, i0_hbm, i1_hbm, w1_hbm, w2_hbm,
                  ia_v, ib_v, win_v):
    c = lax.axis_index("c")
    s = lax.axis_index("s")
    lo = s * SLOTS
    lane = lax.iota(jnp.int32, 16)

    def init(j, _):
        win_v[pl.ds(j * 16, 16)] = jnp.full((16,), -1, jnp.int32)
        return 0
    lax.fori_loop(0, SLOTS // 16, init, 0, unroll=4)

    def batch(kb, loader):
        locs, vals, masks = [], [], []
        for u in range(UB):
            k = kb * UB + u
            iv = loader(k)
            m0 = (iv >= lo) & (iv < lo + SLOTS)
            locs.append(iv - lo)
            vals.append(k * 16 + lane)
            masks.append(m0)
        for u in range(UB):
            plsc.store_scatter(win_v, [locs[u]], vals[u], mask=masks[u])
        anyr = jnp.zeros((16,), jnp.int32)
        retries = []
        for u in range(UB):
            rb = plsc.load_gather(win_v, [locs[u]], mask=masks[u])
            r = masks[u] & (rb < vals[u])
            retries.append(r)
            anyr = anyr | r.astype(jnp.int32)

        @pl.when(jnp.max(anyr) > 0)
        def _fix():
            def _cond(rs):
                t = rs[0].astype(jnp.int32)
                for u in range(1, UB):
                    t = t | rs[u].astype(jnp.int32)
                return jnp.max(t) > 0

            def _body(rs):
                for u in range(UB):
                    plsc.store_scatter(win_v, [locs[u]], vals[u], mask=rs[u])
                out = []
                for u in range(UB):
                    rb2 = plsc.load_gather(win_v, [locs[u]], mask=rs[u])
                    out.append(rs[u] & (rb2 < vals[u]))
                return tuple(out)

            lax.while_loop(_cond, _body, tuple(retries))

    @pl.when(c == 0)
    def _():
        pltpu.sync_copy(fb_hbm, ia_v)
        pltpu.sync_copy(fn_hbm, ib_v)

        def loop0(kb, _):
            def load0(k):
                o = pl.ds(k * 16, 16)
                return ib_v[o] * B + ia_v[o]
            batch(kb, load0)
            return 0
        lax.fori_loop(0, NV // UB, loop0, 0)
        pltpu.sync_copy(win_v, w1_hbm.at[pl.ds(lo, SLOTS)])

    @pl.when(c == 1)
    def _():
        pltpu.sync_copy(i0_hbm, ia_v.at[pl.ds(0, N0)])
        pltpu.sync_copy(i1_hbm, ia_v.at[pl.ds(N0, N1)])

        def loop1(kb, _):
            batch(kb, lambda k: ia_v[pl.ds(k * 16, 16)])
            return 0
        lax.fori_loop(0, NV // UB, loop1, 0)
        pltpu.sync_copy(win_v, w2_hbm.at[pl.ds(lo, SLOTS)])


def _winners(fb, fn, i0, i1):
    mesh = plsc.VectorSubcoreMesh(core_axis_name="c", subcore_axis_name="s")
    fn_k = pl.kernel(
        _winners_body,
        mesh=mesh,
        compiler_params=pltpu.CompilerParams(needs_layout_passes=False),
        out_type=(jax.ShapeDtypeStruct((N,), jnp.int32),
                  jax.ShapeDtypeStruct((N,), jnp.int32)),
        scratch_types=[
            pltpu.VMEM((N,), jnp.int32),
            pltpu.VMEM((N,), jnp.int32),
            pltpu.VMEM((SLOTS,), jnp.int32),
        ],
    )
    return fn_k(fb, fn, i0, i1)


def _srcidx_body(w1_hbm, w2_hbm, src_hbm, w2_v, w1_v, src_v):
    c = lax.axis_index("c")
    s = lax.axis_index("s")
    wid = s * 2 + c
    base = wid * PCHUNK
    lane = lax.iota(jnp.int32, 16)

    pltpu.sync_copy(w2_hbm, w2_v)
    pltpu.sync_copy(w1_hbm.at[pl.ds(base, PCHUNK)], w1_v)

    def comp(j, _):
        w = w1_v[pl.ds(j * 16, 16)]
        pv = base + j * 16 + lane
        cl = jnp.where(w < 0, pv, w)
        sv = plsc.load_gather(w2_v, [cl])
        src = jnp.where((w >= 0) & (sv >= 0), sv, N + (pv & (PAD - 1)))
        src_v[pl.ds(j * 16, 16)] = src
        return 0
    lax.fori_loop(0, PCHUNK // 16, comp, 0)
    pltpu.sync_copy(src_v, src_hbm.at[pl.ds(base, PCHUNK)])


def _srcidx(w1, w2):
    mesh = plsc.VectorSubcoreMesh(core_axis_name="c", subcore_axis_name="s")
    fn_k = pl.kernel(
        _srcidx_body,
        mesh=mesh,
        compiler_params=pltpu.CompilerParams(needs_layout_passes=False),
        out_type=jax.ShapeDtypeStruct((N,), jnp.int32),
        scratch_types=[
            pltpu.VMEM((N,), jnp.int32),
            pltpu.VMEM((PCHUNK,), jnp.int32),
            pltpu.VMEM((PCHUNK,), jnp.int32),
        ],
    )
    return fn_k(w1, w2)


def _gather_body(enc_hbm, src_hbm, out_hbm, src_v, rbuf0_v, rbuf1_v,
                 sem0, sem1):
    c = lax.axis_index("c")
    s = lax.axis_index("s")
    wid = s * 2 + c
    base = wid * PCHUNK

    pltpu.sync_copy(src_hbm.at[pl.ds(base, PCHUNK)], src_v)

    bufs = (rbuf0_v, rbuf1_v)
    sems = (sem0, sem1)

    def _start(g, par):
        pltpu.async_copy(enc_hbm.at[src_v.at[pl.ds(g * GC, GC)]], bufs[par],
                         sems[par])

    def _drain(g, par):
        pltpu.make_async_copy(enc_hbm.at[src_v.at[pl.ds(g * GC, GC)]],
                              bufs[par], sems[par]).wait()
        pltpu.sync_copy(bufs[par], out_hbm.at[pl.ds(base + g * GC, GC)])

    _start(0, 0)

    def gpair(gp, _):
        g = gp * 2
        _start(g + 1, 1)
        _drain(g, 0)

        @pl.when(g + 2 < NGC)
        def _():
            _start(g + 2, 0)
        _drain(g + 1, 1)
        return 0
    lax.fori_loop(0, NGC // 2, gpair, 0)


def _gather_rows(enc_pad, src):
    mesh = plsc.VectorSubcoreMesh(core_axis_name="c", subcore_axis_name="s")
    fn_k = pl.kernel(
        _gather_body,
        mesh=mesh,
        compiler_params=pltpu.CompilerParams(needs_layout_passes=False),
        out_type=jax.ShapeDtypeStruct((N, CH), jnp.float32),
        scratch_types=[
            pltpu.VMEM((PCHUNK,), jnp.int32),
            pltpu.VMEM((GC, CH), jnp.float32),
            pltpu.VMEM((GC, CH), jnp.float32),
            pltpu.SemaphoreType.DMA,
            pltpu.SemaphoreType.DMA,
        ],
    )
    return fn_k(enc_pad, src)


def kernel(feats_t0, feats_t1, idx_t0, idx_t1, flat_batch_idx, flat_nbr_idx,
           neighbor_types, W_in, b_in, W_res, b_res, W_out, b_out):
    enc_pad = _encode_all(feats_t0.T, feats_t1.T, W_in, b_in, W_res, b_res,
                          W_out, b_out)
    w1, w2 = _winners(flat_batch_idx.astype(jnp.int32),
                      flat_nbr_idx.astype(jnp.int32),
                      idx_t0.astype(jnp.int32), idx_t1.astype(jnp.int32))
    src = _srcidx(w1, w2)
    out = _gather_rows(enc_pad, src)
    return out.reshape(K, B, CH).transpose(1, 0, 2)


# speedup vs baseline: 1.2396x; 1.2396x over previous
import functools

import jax
import jax.numpy as jnp
from jax import lax
from jax.experimental import pallas as pl
from jax.experimental.pallas import tpu as pltpu
from jax.experimental.pallas import tpu_sc as plsc

CH = 128
F = 32
B = 1024
K = 50
N = B * K
N0 = N // 2
N1 = N - N0
NUM_RES = 4
PAD = 1024
BLK = 1024
NB0 = N0 // BLK
NBD = N // BLK
NBT = (N + PAD) // BLK

NTILES = 32
UB = 8
SLOTS = N // 16
NV = N // 16
PCHUNK = N // NTILES
GC = 80
NGC = PCHUNK // GC


def _enc_body(x0_ref, x1_ref, win_ref, bin_ref, wres_ref, bres_ref, wout_ref,
              bout_ref, out_ref):
    pid = pl.program_id(0)
    x = jnp.where(pid < NB0, x0_ref[...], x1_ref[...])
    x = jnp.where(jnp.isnan(x), 0.0, x)
    x = jnp.where(x == jnp.inf, 1e6, x)
    x = jnp.where(x == -jnp.inf, -1e6, x)
    dot = functools.partial(jnp.dot, preferred_element_type=jnp.float32)
    w_in = win_ref[0]
    h = jax.nn.relu(
        lax.dot_general(x, w_in, (((0,), (0,)), ((), ())),
                        preferred_element_type=jnp.float32) + bin_ref[0, 0])
    for i in range(NUM_RES):
        h = h + jax.nn.relu(dot(h, wres_ref[0, i]) + bres_ref[0, i])
    y = dot(h, wout_ref[0]) + bout_ref[0, 0]
    out_ref[...] = jnp.where(pid >= NBD, jnp.zeros_like(y), y)


def _encode_all(x0, x1, W_in, b_in, W_res, b_res, W_out, b_out):
    t = lambda i: (i >= NB0).astype(jnp.int32)
    return pl.pallas_call(
        _enc_body,
        grid=(NBT,),
        in_specs=[
            pl.BlockSpec((F, BLK), lambda i: (0, jnp.minimum(i, NB0 - 1))),
            pl.BlockSpec((F, BLK),
                         lambda i: (0, jnp.clip(i - NB0, 0, NBD - NB0 - 1))),
            pl.BlockSpec((1, F, CH), lambda i: (t(i), 0, 0)),
            pl.BlockSpec((1, 1, CH), lambda i: (t(i), 0, 0)),
            pl.BlockSpec((1, NUM_RES, CH, CH), lambda i: (t(i), 0, 0, 0)),
            pl.BlockSpec((1, NUM_RES, CH), lambda i: (t(i), 0, 0)),
            pl.BlockSpec((1, CH, CH), lambda i: (t(i), 0, 0)),
            pl.BlockSpec((1, 1, CH), lambda i: (t(i), 0, 0)),
        ],
        out_specs=pl.BlockSpec((BLK, CH), lambda i: (i, 0)),
        out_shape=jax.ShapeDtypeStruct((N + PAD, CH), jnp.float32),
    )(x0, x1, W_in, b_in.reshape(2, 1, CH), W_res, b_res, W_out,
      b_out.reshape(2, 1, CH))


def _winners_body(fb_hbm, fn_hbm, i0_hbm, i1_hbm, w1_hbm, w2_hbm,
                  ia_v, ib_v, win_v):
    c = lax.axis_index("c")
    s = lax.axis_index("s")
    lo = s * SLOTS
    lane = lax.iota(jnp.int32, 16)

    def init(j, _):
        win_v[pl.ds(j * 16, 16)] = jnp.full((16,), -1, jnp.int32)
        return 0
    lax.fori_loop(0, SLOTS // 16, init, 0, unroll=4)

    def batch(kb, loader):
        locs, vals, masks = [], [], []
        for u in range(UB):
            k = kb * UB + u
            iv = loader(k)
            m0 = (iv >= lo) & (iv < lo + SLOTS)
            locs.append(iv - lo)
            vals.append(k * 16 + lane)
            masks.append(m0)
        for u in range(UB):
            plsc.store_scatter(win_v, [locs[u]], vals[u], mask=masks[u])
        anyr = jnp.zeros((16,), jnp.int32)
        retries = []
        for u in range(UB):
            rb = plsc.load_gather(win_v, [locs[u]], mask=masks[u])
            r = masks[u] & (rb < vals[u])
            retries.append(r)
            anyr = anyr | r.astype(jnp.int32)

        @pl.when(jnp.max(anyr) > 0)
        def _fix():
            def _cond(rs):
                t = rs[0].astype(jnp.int32)
                for u in range(1, UB):
                    t = t | rs[u].astype(jnp.int32)
                return jnp.max(t) > 0

            def _body(rs):
                for u in range(UB):
                    plsc.store_scatter(win_v, [locs[u]], vals[u], mask=rs[u])
                out = []
                for u in range(UB):
                    rb2 = plsc.load_gather(win_v, [locs[u]], mask=rs[u])
                    out.append(rs[u] & (rb2 < vals[u]))
                return tuple(out)

            lax.while_loop(_cond, _body, tuple(retries))

    @pl.when(c == 0)
    def _():
        pltpu.sync_copy(fb_hbm, ia_v)
        pltpu.sync_copy(fn_hbm, ib_v)

        def loop0(kb, _):
            def load0(k):
                o = pl.ds(k * 16, 16)
                return ib_v[o] * B + ia_v[o]
            batch(kb, load0)
            return 0
        lax.fori_loop(0, NV // UB, loop0, 0)
        pltpu.sync_copy(win_v, w1_hbm.at[pl.ds(lo, SLOTS)])

    @pl.when(c == 1)
    def _():
        pltpu.sync_copy(i0_hbm, ia_v.at[pl.ds(0, N0)])
        pltpu.sync_copy(i1_hbm, ia_v.at[pl.ds(N0, N1)])

        def loop1(kb, _):
            batch(kb, lambda k: ia_v[pl.ds(k * 16, 16)])
            return 0
        lax.fori_loop(0, NV // UB, loop1, 0)
        pltpu.sync_copy(win_v, w2_hbm.at[pl.ds(lo, SLOTS)])


def _winners(fb, fn, i0, i1):
    mesh = plsc.VectorSubcoreMesh(core_axis_name="c", subcore_axis_name="s")
    fn_k = pl.kernel(
        _winners_body,
        mesh=mesh,
        compiler_params=pltpu.CompilerParams(needs_layout_passes=False),
        out_type=(jax.ShapeDtypeStruct((N,), jnp.int32),
                  jax.ShapeDtypeStruct((N,), jnp.int32)),
        scratch_types=[
            pltpu.VMEM((N,), jnp.int32),
            pltpu.VMEM((N,), jnp.int32),
            pltpu.VMEM((SLOTS,), jnp.int32),
        ],
    )
    return fn_k(fb, fn, i0, i1)


def _compose_body(enc_hbm, w1_hbm, w2_hbm, out_hbm, w2_v, w1_v, src_v,
                  rbuf0_v, rbuf1_v, sem0, sem1):
    c = lax.axis_index("c")
    s = lax.axis_index("s")
    wid = s * 2 + c
    base = wid * PCHUNK
    lane = lax.iota(jnp.int32, 16)

    pltpu.sync_copy(w2_hbm, w2_v)
    pltpu.sync_copy(w1_hbm.at[pl.ds(base, PCHUNK)], w1_v)

    def comp(j, _):
        w = w1_v[pl.ds(j * 16, 16)]
        pv = base + j * 16 + lane
        cl = jnp.where(w < 0, pv, w)
        sv = plsc.load_gather(w2_v, [cl])
        src = jnp.where((w >= 0) & (sv >= 0), sv, N + (pv & (PAD - 1)))
        src_v[pl.ds(j * 16, 16)] = src
        return 0
    lax.fori_loop(0, PCHUNK // 16, comp, 0)

    bufs = (rbuf0_v, rbuf1_v)
    sems = (sem0, sem1)

    def _start(g, par):
        pltpu.async_copy(enc_hbm.at[src_v.at[pl.ds(g * GC, GC)]], bufs[par],
                         sems[par])

    def _drain(g, par):
        pltpu.make_async_copy(enc_hbm.at[src_v.at[pl.ds(g * GC, GC)]],
                              bufs[par], sems[par]).wait()
        pltpu.sync_copy(bufs[par], out_hbm.at[pl.ds(base + g * GC, GC)])

    _start(0, 0)

    def gpair(gp, _):
        g = gp * 2
        _start(g + 1, 1)
        _drain(g, 0)

        @pl.when(g + 2 < NGC)
        def _():
            _start(g + 2, 0)
        _drain(g + 1, 1)
        return 0
    lax.fori_loop(0, NGC // 2, gpair, 0)


def _compose(enc_pad, w1, w2):
    mesh = plsc.VectorSubcoreMesh(core_axis_name="c", subcore_axis_name="s")
    fn_k = pl.kernel(
        _compose_body,
        mesh=mesh,
        compiler_params=pltpu.CompilerParams(needs_layout_passes=False),
        out_type=jax.ShapeDtypeStruct((N, CH), jnp.float32),
        scratch_types=[
            pltpu.VMEM((N,), jnp.int32),
            pltpu.VMEM((PCHUNK,), jnp.int32),
            pltpu.VMEM((PCHUNK,), jnp.int32),
            pltpu.VMEM((GC, CH), jnp.float32),
            pltpu.VMEM((GC, CH), jnp.float32),
            pltpu.SemaphoreType.DMA,
            pltpu.SemaphoreType.DMA,
        ],
    )
    return fn_k(enc_pad, w1, w2)


def kernel(feats_t0, feats_t1, idx_t0, idx_t1, flat_batch_idx, flat_nbr_idx,
           neighbor_types, W_in, b_in, W_res, b_res, W_out, b_out):
    enc_pad = _encode_all(feats_t0.T, feats_t1.T, W_in, b_in, W_res, b_res,
                          W_out, b_out)
    w1, w2 = _winners(flat_batch_idx.astype(jnp.int32),
                      flat_nbr_idx.astype(jnp.int32),
                      idx_t0.astype(jnp.int32), idx_t1.astype(jnp.int32))
    out = _compose(enc_pad, w1, w2)
    return out.reshape(K, B, CH).transpose(1, 0, 2)


# speedup vs baseline: 1.2447x; 1.0041x over previous
import functools

import jax
import jax.numpy as jnp
from jax import lax
from jax.experimental import pallas as pl
from jax.experimental.pallas import tpu as pltpu
from jax.experimental.pallas import tpu_sc as plsc

CH = 128
F = 32
B = 1024
K = 50
N = B * K
N0 = N // 2
N1 = N - N0
NUM_RES = 4
PAD = 1024
BLK = 1024
NB0 = N0 // BLK
NBD = N // BLK
NBT = (N + PAD) // BLK

NTILES = 32
UB = 8
SLOTS = N // 16
NV = N // 16
PCHUNK = N // NTILES
GC = 80
NGC = PCHUNK // GC


def _enc_body(x0_ref, x1_ref, win_ref, bin_ref, wres_ref, bres_ref, wout_ref,
              bout_ref, out_ref):
    pid = pl.program_id(0)
    x = jnp.where(pid < NB0, x0_ref[...], x1_ref[...])
    x = jnp.where(jnp.isnan(x), 0.0, x)
    x = jnp.where(x == jnp.inf, 1e6, x)
    x = jnp.where(x == -jnp.inf, -1e6, x)
    dot = functools.partial(jnp.dot, preferred_element_type=jnp.float32)
    w_in = win_ref[0]
    h = jax.nn.relu(
        lax.dot_general(x, w_in, (((0,), (0,)), ((), ())),
                        preferred_element_type=jnp.float32) + bin_ref[0, 0])
    for i in range(NUM_RES):
        h = h + jax.nn.relu(dot(h, wres_ref[0, i]) + bres_ref[0, i])
    y = dot(h, wout_ref[0]) + bout_ref[0, 0]
    out_ref[...] = jnp.where(pid >= NBD, jnp.zeros_like(y), y)


def _encode_all(x0, x1, W_in, b_in, W_res, b_res, W_out, b_out):
    t = lambda i: (i >= NB0).astype(jnp.int32)
    return pl.pallas_call(
        _enc_body,
        grid=(NBT,),
        in_specs=[
            pl.BlockSpec((F, BLK), lambda i: (0, jnp.minimum(i, NB0 - 1))),
            pl.BlockSpec((F, BLK),
                         lambda i: (0, jnp.clip(i - NB0, 0, NBD - NB0 - 1))),
            pl.BlockSpec((1, F, CH), lambda i: (t(i), 0, 0)),
            pl.BlockSpec((1, 1, CH), lambda i: (t(i), 0, 0)),
            pl.BlockSpec((1, NUM_RES, CH, CH), lambda i: (t(i), 0, 0, 0)),
            pl.BlockSpec((1, NUM_RES, CH), lambda i: (t(i), 0, 0)),
            pl.BlockSpec((1, CH, CH), lambda i: (t(i), 0, 0)),
            pl.BlockSpec((1, 1, CH), lambda i: (t(i), 0, 0)),
        ],
        out_specs=pl.BlockSpec((BLK, CH), lambda i: (i, 0)),
        out_shape=jax.ShapeDtypeStruct((N + PAD, CH), jnp.float32),
    )(x0, x1, W_in, b_in.reshape(2, 1, CH), W_res, b_res, W_out,
      b_out.reshape(2, 1, CH))


def _winners_body(fb_hbm, fn_hbm, i0_hbm, i1_hbm, w1_hbm, w2_hbm,
                  ia_v, ib_v, win_v):
    c = lax.axis_index("c")
    s = lax.axis_index("s")
    lo = s * SLOTS
    lane = lax.iota(jnp.int32, 16)

    def init(j, _):
        win_v[pl.ds(j * 16, 16)] = jnp.full((16,), -1, jnp.int32)
        return 0
    lax.fori_loop(0, SLOTS // 16, init, 0, unroll=4)

    def batch(kb, loader):
        locs, vals, masks = [], [], []
        for u in range(UB):
            k = kb * UB + u
            iv = loader(k)
            m0 = (iv >= lo) & (iv < lo + SLOTS)
            locs.append(iv - lo)
            vals.append(k * 16 + lane)
            masks.append(m0)
        for u in range(UB):
            plsc.store_scatter(win_v, [locs[u]], vals[u], mask=masks[u])
        anyr = jnp.zeros((16,), jnp.int32)
        retries = []
        for u in range(UB):
            rb = plsc.load_gather(win_v, [locs[u]], mask=masks[u])
            r = masks[u] & (rb < vals[u])
            retries.append(r)
            anyr = anyr | r.astype(jnp.int32)

        @pl.when(jnp.max(anyr) > 0)
        def _fix():
            def _cond(rs):
                t = rs[0].astype(jnp.int32)
                for u in range(1, UB):
                    t = t | rs[u].astype(jnp.int32)
                return jnp.max(t) > 0

            def _body(rs):
                for u in range(UB):
                    plsc.store_scatter(win_v, [locs[u]], vals[u], mask=rs[u])
                out = []
                for u in range(UB):
                    rb2 = plsc.load_gather(win_v, [locs[u]], mask=rs[u])
                    out.append(rs[u] & (rb2 < vals[u]))
                return tuple(out)

            lax.while_loop(_cond, _body, tuple(retries))

    @pl.when(c == 0)
    def _():
        pltpu.sync_copy(fb_hbm, ia_v)
        pltpu.sync_copy(fn_hbm, ib_v)

        def loop0(kb, _):
            def load0(k):
                o = pl.ds(k * 16, 16)
                return ib_v[o] * B + ia_v[o]
            batch(kb, load0)
            return 0
        lax.fori_loop(0, NV // UB, loop0, 0)
        pltpu.sync_copy(win_v, w1_hbm.at[pl.ds(lo, SLOTS)])

    @pl.when(c == 1)
    def _():
        pltpu.sync_copy(i0_hbm, ia_v.at[pl.ds(0, N0)])
        pltpu.sync_copy(i1_hbm, ia_v.at[pl.ds(N0, N1)])

        def loop1(kb, _):
            batch(kb, lambda k: ia_v[pl.ds(k * 16, 16)])
            return 0
        lax.fori_loop(0, NV // UB, loop1, 0)
        pltpu.sync_copy(win_v, w2_hbm.at[pl.ds(lo, SLOTS)])


def _winners(fb, fn, i0, i1):
    mesh = plsc.VectorSubcoreMesh(core_axis_name="c", subcore_axis_name="s")
    fn_k = pl.kernel(
        _winners_body,
        mesh=mesh,
        compiler_params=pltpu.CompilerParams(needs_layout_passes=False),
        out_type=(jax.ShapeDtypeStruct((N,), jnp.int32),
                  jax.ShapeDtypeStruct((N,), jnp.int32)),
        scratch_types=[
            pltpu.VMEM((N,), jnp.int32),
            pltpu.VMEM((N,), jnp.int32),
            pltpu.VMEM((SLOTS,), jnp.int32),
        ],
    )
    return fn_k(fb, fn, i0, i1)


def _compose_body(enc_hbm, w1_hbm, w2_hbm, out_hbm, w2_v, w1_v, src_v,
                  rbuf0_v, rbuf1_v, rbuf2_v, rbuf3_v, sem0, sem1, sem2, sem3):
    c = lax.axis_index("c")
    s = lax.axis_index("s")
    wid = s * 2 + c
    base = wid * PCHUNK
    lane = lax.iota(jnp.int32, 16)

    pltpu.sync_copy(w2_hbm, w2_v)
    pltpu.sync_copy(w1_hbm.at[pl.ds(base, PCHUNK)], w1_v)

    def comp(j, _):
        w = w1_v[pl.ds(j * 16, 16)]
        pv = base + j * 16 + lane
        cl = jnp.where(w < 0, pv, w)
        sv = plsc.load_gather(w2_v, [cl])
        src = jnp.where((w >= 0) & (sv >= 0), sv, N + (pv & (PAD - 1)))
        src_v[pl.ds(j * 16, 16)] = src
        return 0
    lax.fori_loop(0, PCHUNK // 16, comp, 0)

    bufs = (rbuf0_v, rbuf1_v, rbuf2_v, rbuf3_v)
    sems = (sem0, sem1, sem2, sem3)
    DEPTH = 4

    def _start(g, par):
        pltpu.async_copy(enc_hbm.at[src_v.at[pl.ds(g * GC, GC)]], bufs[par],
                         sems[par])

    def _drain(g, par):
        pltpu.make_async_copy(enc_hbm.at[src_v.at[pl.ds(g * GC, GC)]],
                              bufs[par], sems[par]).wait()
        pltpu.sync_copy(bufs[par], out_hbm.at[pl.ds(base + g * GC, GC)])

    for q in range(DEPTH - 1):
        _start(q, q)

    def ggroup(gq, _):
        g = gq * DEPTH
        for q in range(DEPTH):
            @pl.when(g + q + DEPTH - 1 < NGC)
            def _():
                _start(g + q + DEPTH - 1, (q + DEPTH - 1) % DEPTH)
            _drain(g + q, q)
        return 0
    lax.fori_loop(0, NGC // DEPTH, ggroup, 0)


def _compose(enc_pad, w1, w2):
    mesh = plsc.VectorSubcoreMesh(core_axis_name="c", subcore_axis_name="s")
    fn_k = pl.kernel(
        _compose_body,
        mesh=mesh,
        compiler_params=pltpu.CompilerParams(needs_layout_passes=False),
        out_type=jax.ShapeDtypeStruct((N, CH), jnp.float32),
        scratch_types=[
            pltpu.VMEM((N,), jnp.int32),
            pltpu.VMEM((PCHUNK,), jnp.int32),
            pltpu.VMEM((PCHUNK,), jnp.int32),
            pltpu.VMEM((GC, CH), jnp.float32),
            pltpu.VMEM((GC, CH), jnp.float32),
            pltpu.VMEM((GC, CH), jnp.float32),
            pltpu.VMEM((GC, CH), jnp.float32),
            pltpu.SemaphoreType.DMA,
            pltpu.SemaphoreType.DMA,
            pltpu.SemaphoreType.DMA,
            pltpu.SemaphoreType.DMA,
        ],
    )
    return fn_k(enc_pad, w1, w2)


def kernel(feats_t0, feats_t1, idx_t0, idx_t1, flat_batch_idx, flat_nbr_idx,
           neighbor_types, W_in, b_in, W_res, b_res, W_out, b_out):
    enc_pad = _encode_all(feats_t0.T, feats_t1.T, W_in, b_in, W_res, b_res,
                          W_out, b_out)
    w1, w2 = _winners(flat_batch_idx.astype(jnp.int32),
                      flat_nbr_idx.astype(jnp.int32),
                      idx_t0.astype(jnp.int32), idx_t1.astype(jnp.int32))
    out = _compose(enc_pad, w1, w2)
    return out.reshape(K, B, CH).transpose(1, 0, 2)
